# baseline (speedup 1.0000x reference)
import functools

import jax
import jax.numpy as jnp
from jax import lax
from jax.experimental import pallas as pl
from jax.experimental.pallas import tpu as pltpu
from jax.experimental.pallas import tpu_sc as plsc

_PS_DIM = 64
_TABLE_ROWS = 16
_NC = 2
_NS = 16
_NW = _NC * _NS
_IPW = 128
_K = 5
_CH = _K * _IPW


def _sc_lookup(table, idx3, rows_per_w):
    mesh = plsc.VectorSubcoreMesh(core_axis_name="c", subcore_axis_name="s")
    n_rows = _NW * rows_per_w
    n_chunks = rows_per_w // _CH

    @functools.partial(
        pl.kernel,
        out_type=jax.ShapeDtypeStruct((n_rows, _PS_DIM), jnp.float32),
        mesh=mesh,
        scratch_types=[
            pltpu.VMEM_SHARED((_TABLE_ROWS, _PS_DIM), jnp.float32),
            pltpu.VMEM((rows_per_w // _IPW, _IPW), jnp.int32),
            pltpu.VMEM((_CH, _PS_DIM), jnp.float32),
            pltpu.VMEM((_CH, _PS_DIM), jnp.float32),
            pltpu.SemaphoreType.DMA,
            pltpu.SemaphoreType.DMA,
            pltpu.SemaphoreType.DMA,
        ],
        compiler_params=pltpu.CompilerParams(
            use_tc_tiling_on_sc=False, needs_layout_passes=False
        ),
    )
    def k(table_hbm, idx_hbm, out_hbm, table_sh, idx_v, buf0, buf1,
          gsem, sem0, sem1):
        sid = lax.axis_index("s")
        wid = sid * _NC + lax.axis_index("c")

        @pl.when(sid == 0)
        def _():
            pltpu.sync_copy(table_hbm, table_sh)

        pltpu.sync_copy(idx_hbm.at[wid], idx_v)
        plsc.subcore_barrier()
        base = wid * rows_per_w

        def fire(buf, chunk):
            for i in range(_K):
                pltpu.async_copy(
                    table_sh.at[idx_v.at[chunk * _K + i]],
                    buf.at[pl.ds(i * _IPW, _IPW)],
                    gsem,
                )

        def wait_fire(buf):
            for i in range(_K):
                pltpu.make_async_copy(
                    table_sh.at[idx_v.at[i]],
                    buf.at[pl.ds(i * _IPW, _IPW)],
                    gsem,
                ).wait()

        def flush(buf, sem, chunk):
            pltpu.async_copy(
                buf, out_hbm.at[pl.ds(base + chunk * _CH, _CH)], sem
            )

        def drain(buf, sem):
            pltpu.make_async_copy(out_hbm.at[pl.ds(base, _CH)], buf, sem).wait()

        fire(buf0, 0)
        wait_fire(buf0)
        flush(buf0, sem0, 0)
        fire(buf1, 1)

        def outer(g2, carry):
            ko = g2 * 2 - 1
            wait_fire(buf1)
            flush(buf1, sem1, ko)
            drain(buf0, sem0)
            fire(buf0, ko + 1)
            wait_fire(buf0)
            flush(buf0, sem0, ko + 1)
            drain(buf1, sem1)
            fire(buf1, ko + 2)
            return carry

        lax.fori_loop(1, n_chunks // 2, outer, 0)
        wait_fire(buf1)
        flush(buf1, sem1, n_chunks - 1)
        drain(buf0, sem0)
        drain(buf1, sem1)

    return k(table, idx3)


def kernel(batch_rgn_sqn, encoding):
    b, l = batch_rgn_sqn.shape
    n = b * l
    rows_per_w = n // _NW
    assert rows_per_w % _CH == 0
    table = encoding[:_TABLE_ROWS, :_PS_DIM]
    idx3 = batch_rgn_sqn.astype(jnp.int32).reshape(_NW, rows_per_w // _IPW, _IPW)
    out = _sc_lookup(table, idx3, rows_per_w)
    return out.reshape(b, l, _PS_DIM)
